# baseline (speedup 1.0000x reference)
import functools

import jax
import jax.numpy as jnp
from jax import lax
from jax.experimental import pallas as pl
from jax.experimental.pallas import tpu as pltpu
from jax.experimental.pallas import tpu_sc as plsc

N = 10000
D = 128
NPAD = 10240
NC, NS, L = 2, 16, 16
NW = NC * NS
K = 128
KA = 64
NBUF = 3
ROWS_PER_TILE = NPAD // NS


def _sc_mesh():
    return plsc.VectorSubcoreMesh(core_axis_name="c", subcore_axis_name="s")


def _make_deg_kernel(nb):
    @functools.partial(
        pl.kernel,
        out_type=jax.ShapeDtypeStruct((NC, 2, NPAD), jnp.float32),
        mesh=_sc_mesh(),
        scratch_types=[
            pltpu.VMEM((nb, 2, K), jnp.int32),
            pltpu.VMEM((nb, K), jnp.float32),
            pltpu.VMEM((ROWS_PER_TILE,), jnp.float32),
            pltpu.VMEM_SHARED((NPAD,), jnp.float32),
            pltpu.VMEM_SHARED((NPAD,), jnp.float32),
        ],
    )
    def k(sd_hbm, w_hbm, out_hbm, sd_v, w_v, zero_v, dego_sp, degi_sp):
        cid = lax.axis_index("c")
        sid = lax.axis_index("s")
        wid = cid * NS + sid

        zero = jnp.zeros((L,), jnp.float32)

        def zloop(i, _):
            zero_v[pl.ds(i * L, L)] = zero
            return 0

        lax.fori_loop(0, ROWS_PER_TILE // L, zloop, 0)
        pltpu.sync_copy(zero_v,
                        dego_sp.at[pl.ds(sid * ROWS_PER_TILE, ROWS_PER_TILE)])
        pltpu.sync_copy(zero_v,
                        degi_sp.at[pl.ds(sid * ROWS_PER_TILE, ROWS_PER_TILE)])
        pltpu.sync_copy(sd_hbm.at[wid], sd_v)
        pltpu.sync_copy(w_hbm.at[wid], w_v)
        plsc.subcore_barrier()

        def body(b, _):
            pltpu.sync_copy(w_v.at[b], dego_sp.at[sd_v.at[b, 0]], add=True)
            pltpu.sync_copy(w_v.at[b], degi_sp.at[sd_v.at[b, 1]], add=True)
            return 0

        lax.fori_loop(0, nb, body, 0)
        plsc.subcore_barrier()

        @pl.when(sid == 0)
        def _():
            pltpu.sync_copy(dego_sp, out_hbm.at[cid, 0])
            pltpu.sync_copy(degi_sp, out_hbm.at[cid, 1])

    return k


def _make_agg_kernel(nb):
    @functools.partial(
        pl.kernel,
        out_type=jax.ShapeDtypeStruct((NC, NPAD, D), jnp.float32),
        mesh=_sc_mesh(),
        scratch_types=[
            pltpu.VMEM((nb, K), jnp.int32),
            pltpu.VMEM((nb, K), jnp.int32),
            pltpu.VMEM((nb, K), jnp.float32),
            pltpu.VMEM((K, D), jnp.float32),
            pltpu.VMEM_SHARED((NPAD, D), jnp.float32),
            pltpu.SemaphoreType.DMA,
        ],
    )
    def k(h_hbm, src_hbm, dst_hbm, w_hbm, out_hbm, src_v, dst_v, w_v,
          rows_v, agg_sp, sem):
        cid = lax.axis_index("c")
        sid = lax.axis_index("s")
        wid = cid * NS + sid

        zero = jnp.zeros((L,), jnp.float32)

        def zloop(r, _):
            for f in range(D // L):
                rows_v[r, pl.ds(f * L, L)] = zero
            return 0

        lax.fori_loop(0, K, zloop, 0)
        for i in range(ROWS_PER_TILE // K):
            pltpu.sync_copy(
                rows_v, agg_sp.at[pl.ds(sid * ROWS_PER_TILE + i * K, K)])
        pltpu.sync_copy(src_hbm.at[wid], src_v)
        pltpu.sync_copy(dst_hbm.at[wid], dst_v)
        pltpu.sync_copy(w_hbm.at[wid], w_v)
        plsc.subcore_barrier()

        gdn = lax.GatherDimensionNumbers(
            offset_dims=(), collapsed_slice_dims=(0,), start_index_map=(0,))

        def body(b, _):
            pltpu.async_copy(h_hbm.at[src_v.at[b]], rows_v, sem).wait()

            def scale(gr, _):
                w16 = w_v[b, pl.ds(gr * L, L)]
                for j in range(L):
                    wb = lax.gather(
                        w16, jnp.full((L, 1), j, jnp.int32), gdn,
                        slice_sizes=(1,),
                        mode=lax.GatherScatterMode.PROMISE_IN_BOUNDS)
                    r = gr * L + j
                    for f in range(D // L):
                        rows_v[r, pl.ds(f * L, L)] = (
                            rows_v[r, pl.ds(f * L, L)] * wb)
                return 0

            lax.fori_loop(0, K // L, scale, 0)
            pltpu.sync_copy(rows_v, agg_sp.at[dst_v.at[b]], add=True)
            return 0

        lax.fori_loop(0, nb, body, 0)
        plsc.subcore_barrier()
        pltpu.sync_copy(
            agg_sp.at[pl.ds(sid * ROWS_PER_TILE, ROWS_PER_TILE)],
            out_hbm.at[cid, pl.ds(sid * ROWS_PER_TILE, ROWS_PER_TILE)])

    return k


BLK = 2000


def _dinv(ref):
    return lax.rsqrt(jnp.maximum(ref[:, 0:1] + ref[:, 1:2], 1e-12))


def _tc_first_body(x_ref, w_ref, go_ref, o_ref):
    h = jnp.dot(x_ref[...], w_ref[...], preferred_element_type=jnp.float32)
    o_ref[...] = h * _dinv(go_ref)


def _tc_first(x, w, dego):
    return pl.pallas_call(
        _tc_first_body,
        out_shape=jax.ShapeDtypeStruct((N, D), jnp.float32),
        grid=(N // BLK,),
        in_specs=[
            pl.BlockSpec((BLK, D), lambda i: (i, 0)),
            pl.BlockSpec((D, D), lambda i: (0, 0)),
            pl.BlockSpec((BLK, 2), lambda i: (i, 0)),
        ],
        out_specs=pl.BlockSpec((BLK, D), lambda i: (i, 0)),
    )(x, w, dego)


def _tc_mid_body(p_ref, gi_ref, go_ref, b_ref, w_ref, o_ref):
    agg = (p_ref[0] + p_ref[1]) * _dinv(gi_ref)
    x = jnp.maximum(agg + b_ref[...], 0.0)
    h = jnp.dot(x, w_ref[...], preferred_element_type=jnp.float32)
    o_ref[...] = h * _dinv(go_ref)


def _tc_mid(parts, degi, dego, b, w):
    return pl.pallas_call(
        _tc_mid_body,
        out_shape=jax.ShapeDtypeStruct((N, D), jnp.float32),
        grid=(N // BLK,),
        in_specs=[
            pl.BlockSpec((NC, BLK, D), lambda i: (0, i, 0)),
            pl.BlockSpec((BLK, 2), lambda i: (i, 0)),
            pl.BlockSpec((BLK, 2), lambda i: (i, 0)),
            pl.BlockSpec((1, D), lambda i: (0, 0)),
            pl.BlockSpec((D, D), lambda i: (0, 0)),
        ],
        out_specs=pl.BlockSpec((BLK, D), lambda i: (i, 0)),
    )(parts, degi, dego, b, w)


def _tc_final_body(p_ref, gi_ref, b_ref, o_ref):
    agg = (p_ref[0] + p_ref[1]) * _dinv(gi_ref)
    x = jnp.maximum(agg + b_ref[...], 0.0)
    m = jnp.max(x, axis=1, keepdims=True)
    e = jnp.exp(x - m)
    o_ref[...] = e / jnp.sum(e, axis=1, keepdims=True)


def _tc_final(parts, degi, b):
    return pl.pallas_call(
        _tc_final_body,
        out_shape=jax.ShapeDtypeStruct((N, D), jnp.float32),
        grid=(N // BLK,),
        in_specs=[
            pl.BlockSpec((NC, BLK, D), lambda i: (0, i, 0)),
            pl.BlockSpec((BLK, 2), lambda i: (i, 0)),
            pl.BlockSpec((1, D), lambda i: (0, 0)),
        ],
        out_specs=pl.BlockSpec((BLK, D), lambda i: (i, 0)),
    )(parts, degi, b)


def kernel(in_feat, edge_index, edge_weight, W0, b0, W1, b1, W2, b2):
    src = edge_index[0]
    dst = edge_index[1]
    e = edge_weight.shape[0]
    nb = -(-e // (NW * K))
    nb += (-nb) % 4
    epad = NW * nb * K - e

    def pad(x):
        return jnp.pad(x, (0, epad)).reshape(NW, nb, 1, K)

    sdpad = jnp.concatenate([pad(src), pad(dst)], axis=2)
    wpad = jnp.pad(edge_weight, (0, epad)).reshape(NW, nb, K)
    deg = _make_deg_kernel(nb)(sdpad, wpad)

    srca = sdpad[:, :, 0]
    dsta = sdpad[:, :, 1]
    wa = wpad
    dego = jnp.stack([deg[0, 0], deg[1, 0]], axis=1)
    degi = jnp.stack([deg[0, 1], deg[1, 1]], axis=1)
    b0r = b0.reshape(1, D)
    b1r = b1.reshape(1, D)
    b2r = b2.reshape(1, D)

    agg_k = _make_agg_kernel(nb)
    h = _tc_first(in_feat, W0, dego)
    p = agg_k(h, srca, dsta, wa)
    h = _tc_mid(p, degi, dego, b0r, W1)
    p = agg_k(h, srca, dsta, wa)
    h = _tc_mid(p, degi, dego, b1r, W2)
    p = agg_k(h, srca, dsta, wa)
    return _tc_final(p, degi, b2r)


# speedup vs baseline: 1.4411x; 1.4411x over previous
import functools

import jax
import jax.numpy as jnp
from jax import lax
from jax.experimental import pallas as pl
from jax.experimental.pallas import tpu as pltpu
from jax.experimental.pallas import tpu_sc as plsc

N = 10000
D = 128
NPAD = 10240
NC, NS, L = 2, 16, 16
NW = NC * NS
K = 128
KA = 64
NBUF = 3
ROWS_PER_TILE = NPAD // NS


def _sc_mesh():
    return plsc.VectorSubcoreMesh(core_axis_name="c", subcore_axis_name="s")


def _make_deg_kernel(nb):
    @functools.partial(
        pl.kernel,
        out_type=jax.ShapeDtypeStruct((NC, 2, NPAD), jnp.float32),
        mesh=_sc_mesh(),
        scratch_types=[
            pltpu.VMEM((nb, 2, K), jnp.int32),
            pltpu.VMEM((nb, K), jnp.float32),
            pltpu.VMEM((ROWS_PER_TILE,), jnp.float32),
            pltpu.VMEM_SHARED((NPAD,), jnp.float32),
            pltpu.VMEM_SHARED((NPAD,), jnp.float32),
        ],
    )
    def k(sd_hbm, w_hbm, out_hbm, sd_v, w_v, zero_v, dego_sp, degi_sp):
        cid = lax.axis_index("c")
        sid = lax.axis_index("s")
        wid = cid * NS + sid

        zero = jnp.zeros((L,), jnp.float32)

        def zloop(i, _):
            zero_v[pl.ds(i * L, L)] = zero
            return 0

        lax.fori_loop(0, ROWS_PER_TILE // L, zloop, 0)
        pltpu.sync_copy(zero_v,
                        dego_sp.at[pl.ds(sid * ROWS_PER_TILE, ROWS_PER_TILE)])
        pltpu.sync_copy(zero_v,
                        degi_sp.at[pl.ds(sid * ROWS_PER_TILE, ROWS_PER_TILE)])
        pltpu.sync_copy(sd_hbm.at[wid], sd_v)
        pltpu.sync_copy(w_hbm.at[wid], w_v)
        plsc.subcore_barrier()

        def body(b, _):
            pltpu.sync_copy(w_v.at[b], dego_sp.at[sd_v.at[b, 0]], add=True)
            pltpu.sync_copy(w_v.at[b], degi_sp.at[sd_v.at[b, 1]], add=True)
            return 0

        lax.fori_loop(0, nb, body, 0)
        plsc.subcore_barrier()

        @pl.when(sid == 0)
        def _():
            pltpu.sync_copy(dego_sp, out_hbm.at[cid, 0])
            pltpu.sync_copy(degi_sp, out_hbm.at[cid, 1])

    return k


def _make_agg_kernel(nb):
    @functools.partial(
        pl.kernel,
        out_type=jax.ShapeDtypeStruct((NC, NPAD, D), jnp.float32),
        mesh=_sc_mesh(),
        scratch_types=[
            pltpu.VMEM((nb, K), jnp.int32),
            pltpu.VMEM((nb, K), jnp.int32),
            pltpu.VMEM((nb, K), jnp.float32),
            pltpu.VMEM((K, D), jnp.float32),
            pltpu.VMEM_SHARED((NPAD, D), jnp.float32),
            pltpu.SemaphoreType.DMA,
        ],
    )
    def k(h_hbm, src_hbm, dst_hbm, w_hbm, out_hbm, src_v, dst_v, w_v,
          rows_v, agg_sp, sem):
        cid = lax.axis_index("c")
        sid = lax.axis_index("s")
        wid = cid * NS + sid

        zero = jnp.zeros((L,), jnp.float32)

        def zloop(r, _):
            for f in range(D // L):
                rows_v[r, pl.ds(f * L, L)] = zero
            return 0

        lax.fori_loop(0, K, zloop, 0)
        for i in range(ROWS_PER_TILE // K):
            pltpu.sync_copy(
                rows_v, agg_sp.at[pl.ds(sid * ROWS_PER_TILE + i * K, K)])
        pltpu.sync_copy(src_hbm.at[wid], src_v)
        pltpu.sync_copy(dst_hbm.at[wid], dst_v)
        pltpu.sync_copy(w_hbm.at[wid], w_v)
        plsc.subcore_barrier()

        gdn = lax.GatherDimensionNumbers(
            offset_dims=(), collapsed_slice_dims=(0,), start_index_map=(0,))

        def body(b, _):
            pltpu.async_copy(h_hbm.at[src_v.at[b]], rows_v, sem).wait()

            def scale(gr, _):
                w16 = w_v[b, pl.ds(gr * L, L)]
                for j in range(L):
                    wb = lax.gather(
                        w16, jnp.full((L, 1), j, jnp.int32), gdn,
                        slice_sizes=(1,),
                        mode=lax.GatherScatterMode.PROMISE_IN_BOUNDS)
                    r = gr * L + j
                    for f in range(D // L):
                        rows_v[r, pl.ds(f * L, L)] = (
                            rows_v[r, pl.ds(f * L, L)] * wb)
                return 0

            lax.fori_loop(0, K // L, scale, 0)
            pltpu.sync_copy(rows_v, agg_sp.at[dst_v.at[b]], add=True)
            return 0

        lax.fori_loop(0, nb, body, 0)
        plsc.subcore_barrier()
        pltpu.sync_copy(
            agg_sp.at[pl.ds(sid * ROWS_PER_TILE, ROWS_PER_TILE)],
            out_hbm.at[cid, pl.ds(sid * ROWS_PER_TILE, ROWS_PER_TILE)])

    return k


BLK = 2000


def _dinv(ref):
    return lax.rsqrt(jnp.maximum(ref[:, 0:1] + ref[:, 1:2], 1e-12))


def _tc_first_body(x_ref, w_ref, go_ref, o_ref):
    h = jnp.dot(x_ref[...], w_ref[...], preferred_element_type=jnp.float32)
    o_ref[...] = h * _dinv(go_ref)


def _tc_first(x, w, dego):
    return pl.pallas_call(
        _tc_first_body,
        out_shape=jax.ShapeDtypeStruct((N, D), jnp.float32),
        grid=(N // BLK,),
        in_specs=[
            pl.BlockSpec((BLK, D), lambda i: (i, 0)),
            pl.BlockSpec((D, D), lambda i: (0, 0)),
            pl.BlockSpec((BLK, 2), lambda i: (i, 0)),
        ],
        out_specs=pl.BlockSpec((BLK, D), lambda i: (i, 0)),
    )(x, w, dego)


def _tc_mid_body(p_ref, gi_ref, go_ref, b_ref, w_ref, o_ref):
    agg = (p_ref[0] + p_ref[1]) * _dinv(gi_ref)
    x = jnp.maximum(agg + b_ref[...], 0.0)
    h = jnp.dot(x, w_ref[...], preferred_element_type=jnp.float32)
    o_ref[...] = h * _dinv(go_ref)


def _tc_mid(parts, degi, dego, b, w):
    return pl.pallas_call(
        _tc_mid_body,
        out_shape=jax.ShapeDtypeStruct((N, D), jnp.float32),
        grid=(N // BLK,),
        in_specs=[
            pl.BlockSpec((NC, BLK, D), lambda i: (0, i, 0)),
            pl.BlockSpec((BLK, 2), lambda i: (i, 0)),
            pl.BlockSpec((BLK, 2), lambda i: (i, 0)),
            pl.BlockSpec((1, D), lambda i: (0, 0)),
            pl.BlockSpec((D, D), lambda i: (0, 0)),
        ],
        out_specs=pl.BlockSpec((BLK, D), lambda i: (i, 0)),
    )(parts, degi, dego, b, w)


def _tc_final_body(p_ref, gi_ref, b_ref, o_ref):
    agg = (p_ref[0] + p_ref[1]) * _dinv(gi_ref)
    x = jnp.maximum(agg + b_ref[...], 0.0)
    m = jnp.max(x, axis=1, keepdims=True)
    e = jnp.exp(x - m)
    o_ref[...] = e / jnp.sum(e, axis=1, keepdims=True)


def _tc_final(parts, degi, b):
    return pl.pallas_call(
        _tc_final_body,
        out_shape=jax.ShapeDtypeStruct((N, D), jnp.float32),
        grid=(N // BLK,),
        in_specs=[
            pl.BlockSpec((NC, BLK, D), lambda i: (0, i, 0)),
            pl.BlockSpec((BLK, 2), lambda i: (i, 0)),
            pl.BlockSpec((1, D), lambda i: (0, 0)),
        ],
        out_specs=pl.BlockSpec((BLK, D), lambda i: (i, 0)),
    )(parts, degi, b)


def kernel(in_feat, edge_index, edge_weight, W0, b0, W1, b1, W2, b2):
    src = edge_index[0]
    dst = edge_index[1]
    e = edge_weight.shape[0]
    nb = -(-e // (NW * K))
    epad = NW * nb * K - e

    def pad(x):
        return jnp.pad(x, (0, epad)).reshape(NW, nb, K)

    srca = pad(src)
    dsta = pad(dst)
    wa = pad(edge_weight)
    sdpad = jnp.stack([srca, dsta], axis=2)
    deg = _make_deg_kernel(nb)(sdpad, wa)
    dego = jnp.stack([deg[0, 0], deg[1, 0]], axis=1)
    degi = jnp.stack([deg[0, 1], deg[1, 1]], axis=1)
    b0r = b0.reshape(1, D)
    b1r = b1.reshape(1, D)
    b2r = b2.reshape(1, D)

    agg_k = _make_agg_kernel(nb)
    h = _tc_first(in_feat, W0, dego)
    p = agg_k(h, srca, dsta, wa)
    h = _tc_mid(p, degi, dego, b0r, W1)
    p = agg_k(h, srca, dsta, wa)
    h = _tc_mid(p, degi, dego, b1r, W2)
    p = agg_k(h, srca, dsta, wa)
    return _tc_final(p, degi, b2r)


# speedup vs baseline: 1.6976x; 1.1780x over previous
import functools

import jax
import jax.numpy as jnp
from jax import lax
from jax.experimental import pallas as pl
from jax.experimental.pallas import tpu as pltpu
from jax.experimental.pallas import tpu_sc as plsc

N = 10000
D = 128
NPAD = 10240
NC, NS, L = 2, 16, 16
NW = NC * NS
K = 128
KA = 64
NB0, NB1 = 104, 54
ROWS_PER_TILE = NPAD // NS


def _sc_mesh():
    return plsc.VectorSubcoreMesh(core_axis_name="c", subcore_axis_name="s")


def _make_deg_kernel(nb):
    @functools.partial(
        pl.kernel,
        out_type=jax.ShapeDtypeStruct((NC, 2, NPAD), jnp.float32),
        mesh=_sc_mesh(),
        scratch_types=[
            pltpu.VMEM((nb, 2, K), jnp.int32),
            pltpu.VMEM((nb, K), jnp.float32),
            pltpu.VMEM((ROWS_PER_TILE,), jnp.float32),
            pltpu.VMEM_SHARED((NPAD,), jnp.float32),
            pltpu.VMEM_SHARED((NPAD,), jnp.float32),
        ],
    )
    def k(sd_hbm, w_hbm, out_hbm, sd_v, w_v, zero_v, dego_sp, degi_sp):
        cid = lax.axis_index("c")
        sid = lax.axis_index("s")
        wid = cid * NS + sid

        zero = jnp.zeros((L,), jnp.float32)

        def zloop(i, _):
            zero_v[pl.ds(i * L, L)] = zero
            return 0

        lax.fori_loop(0, ROWS_PER_TILE // L, zloop, 0)
        pltpu.sync_copy(zero_v,
                        dego_sp.at[pl.ds(sid * ROWS_PER_TILE, ROWS_PER_TILE)])
        pltpu.sync_copy(zero_v,
                        degi_sp.at[pl.ds(sid * ROWS_PER_TILE, ROWS_PER_TILE)])
        pltpu.sync_copy(sd_hbm.at[wid], sd_v)
        pltpu.sync_copy(w_hbm.at[wid], w_v)
        plsc.subcore_barrier()

        def body(b, _):
            pltpu.sync_copy(w_v.at[b], dego_sp.at[sd_v.at[b, 0]], add=True)
            pltpu.sync_copy(w_v.at[b], degi_sp.at[sd_v.at[b, 1]], add=True)
            return 0

        lax.fori_loop(0, nb, body, 0)
        plsc.subcore_barrier()

        @pl.when(sid == 0)
        def _():
            pltpu.sync_copy(dego_sp, out_hbm.at[cid, 0])
            pltpu.sync_copy(degi_sp, out_hbm.at[cid, 1])

    return k


def _make_agg_kernel(nb):
    nbmax, nb0, nb1 = nb

    @functools.partial(
        pl.kernel,
        out_type=jax.ShapeDtypeStruct((NC, NPAD, D), jnp.float32),
        mesh=_sc_mesh(),
        scratch_types=[
            pltpu.VMEM((nbmax, K), jnp.int32),
            pltpu.VMEM((nbmax, K), jnp.float32),
            pltpu.VMEM((K,), jnp.int32),
            pltpu.VMEM((K,), jnp.int32),
            pltpu.VMEM((K, D), jnp.float32),
            pltpu.VMEM_SHARED((NPAD, D), jnp.float32),
            pltpu.SemaphoreType.DMA,
        ],
    )
    def k(h_hbm, sd_hbm, w_hbm, out_hbm, sd_v, w_v, src_v, dst_v,
          rows_v, agg_sp, sem):
        cid = lax.axis_index("c")
        sid = lax.axis_index("s")
        wid = cid * NS + sid
        mynb = jnp.where(cid == 0, nb0, nb1)

        zero = jnp.zeros((L,), jnp.float32)

        def zloop(r, _):
            for f in range(D // L):
                rows_v[r, pl.ds(f * L, L)] = zero
            return 0

        lax.fori_loop(0, K, zloop, 0)
        for i in range(ROWS_PER_TILE // K):
            pltpu.sync_copy(
                rows_v, agg_sp.at[pl.ds(sid * ROWS_PER_TILE + i * K, K)])
        pltpu.sync_copy(sd_hbm.at[wid], sd_v)
        pltpu.sync_copy(w_hbm.at[wid], w_v)
        plsc.subcore_barrier()

        gdn = lax.GatherDimensionNumbers(
            offset_dims=(), collapsed_slice_dims=(0,), start_index_map=(0,))

        def body(b, _):
            for g in range(K // L):
                sd16 = sd_v[b, pl.ds(g * L, L)]
                src_v[pl.ds(g * L, L)] = jnp.bitwise_and(
                    sd16, jnp.full((L,), 0x3FFF, jnp.int32))
                dst_v[pl.ds(g * L, L)] = jnp.right_shift(
                    sd16, jnp.full((L,), 14, jnp.int32))
            pltpu.async_copy(h_hbm.at[src_v], rows_v, sem).wait()

            def scale(gr, _):
                w16 = w_v[b, pl.ds(gr * L, L)]
                for j in range(L):
                    wb = lax.gather(
                        w16, jnp.full((L, 1), j, jnp.int32), gdn,
                        slice_sizes=(1,),
                        mode=lax.GatherScatterMode.PROMISE_IN_BOUNDS)
                    r = gr * L + j
                    for f in range(D // L):
                        rows_v[r, pl.ds(f * L, L)] = (
                            rows_v[r, pl.ds(f * L, L)] * wb)
                return 0

            lax.fori_loop(0, K // L, scale, 0)
            pltpu.sync_copy(rows_v, agg_sp.at[dst_v], add=True)
            return 0

        lax.fori_loop(0, mynb, body, 0)
        plsc.subcore_barrier()
        pltpu.sync_copy(
            agg_sp.at[pl.ds(sid * ROWS_PER_TILE, ROWS_PER_TILE)],
            out_hbm.at[cid, pl.ds(sid * ROWS_PER_TILE, ROWS_PER_TILE)])

    return k


BLK = 2000


def _dinv(ref):
    return lax.rsqrt(jnp.maximum(ref[:, 0:1] + ref[:, 1:2], 1e-12))


def _tc_first_body(x_ref, w_ref, go_ref, o_ref):
    h = jnp.dot(x_ref[...], w_ref[...], preferred_element_type=jnp.float32)
    o_ref[...] = h * _dinv(go_ref)


def _tc_first(x, w, dego):
    return pl.pallas_call(
        _tc_first_body,
        out_shape=jax.ShapeDtypeStruct((N, D), jnp.float32),
        grid=(N // BLK,),
        in_specs=[
            pl.BlockSpec((BLK, D), lambda i: (i, 0)),
            pl.BlockSpec((D, D), lambda i: (0, 0)),
            pl.BlockSpec((BLK, 2), lambda i: (i, 0)),
        ],
        out_specs=pl.BlockSpec((BLK, D), lambda i: (i, 0)),
    )(x, w, dego)


def _tc_mid_body(p_ref, gi_ref, go_ref, b_ref, w_ref, o_ref):
    agg = (p_ref[0] + p_ref[1]) * _dinv(gi_ref)
    x = jnp.maximum(agg + b_ref[...], 0.0)
    h = jnp.dot(x, w_ref[...], preferred_element_type=jnp.float32)
    o_ref[...] = h * _dinv(go_ref)


def _tc_mid(parts, degi, dego, b, w):
    return pl.pallas_call(
        _tc_mid_body,
        out_shape=jax.ShapeDtypeStruct((N, D), jnp.float32),
        grid=(N // BLK,),
        in_specs=[
            pl.BlockSpec((NC, BLK, D), lambda i: (0, i, 0)),
            pl.BlockSpec((BLK, 2), lambda i: (i, 0)),
            pl.BlockSpec((BLK, 2), lambda i: (i, 0)),
            pl.BlockSpec((1, D), lambda i: (0, 0)),
            pl.BlockSpec((D, D), lambda i: (0, 0)),
        ],
        out_specs=pl.BlockSpec((BLK, D), lambda i: (i, 0)),
    )(parts, degi, dego, b, w)


def _tc_final_body(p_ref, gi_ref, b_ref, o_ref):
    agg = (p_ref[0] + p_ref[1]) * _dinv(gi_ref)
    x = jnp.maximum(agg + b_ref[...], 0.0)
    m = jnp.max(x, axis=1, keepdims=True)
    e = jnp.exp(x - m)
    o_ref[...] = e / jnp.sum(e, axis=1, keepdims=True)


def _tc_final(parts, degi, b):
    return pl.pallas_call(
        _tc_final_body,
        out_shape=jax.ShapeDtypeStruct((N, D), jnp.float32),
        grid=(N // BLK,),
        in_specs=[
            pl.BlockSpec((NC, BLK, D), lambda i: (0, i, 0)),
            pl.BlockSpec((BLK, 2), lambda i: (i, 0)),
            pl.BlockSpec((1, D), lambda i: (0, 0)),
        ],
        out_specs=pl.BlockSpec((BLK, D), lambda i: (i, 0)),
    )(parts, degi, b)


def kernel(in_feat, edge_index, edge_weight, W0, b0, W1, b1, W2, b2):
    src = edge_index[0]
    dst = edge_index[1]
    e = edge_weight.shape[0]
    nb = -(-e // (NW * K))
    epad = NW * nb * K - e

    def pad(x):
        return jnp.pad(x, (0, epad)).reshape(NW, nb, K)

    srca = pad(src)
    dsta = pad(dst)
    wa = pad(edge_weight)
    sdpad = jnp.stack([srca, dsta], axis=2)
    deg = _make_deg_kernel(nb)(sdpad, wa)

    e0 = NS * NB0 * K
    packed = jnp.bitwise_or(src, jnp.left_shift(dst, 14))
    pk = jnp.pad(packed, (0, NS * (NB0 + NB1) * K - e))
    wk = jnp.pad(edge_weight, (0, NS * (NB0 + NB1) * K - e))
    NBMAX = max(NB0, NB1)

    def split(x):
        a = x[:e0].reshape(NS, NB0, K)
        b = x[e0:].reshape(NS, NB1, K)
        a = jnp.pad(a, ((0, 0), (0, NBMAX - NB0), (0, 0)))
        b = jnp.pad(b, ((0, 0), (0, NBMAX - NB1), (0, 0)))
        return jnp.concatenate([a, b], axis=0)

    sdk = split(pk)
    wkk = split(wk)
    dego = jnp.stack([deg[0, 0], deg[1, 0]], axis=1)
    degi = jnp.stack([deg[0, 1], deg[1, 1]], axis=1)
    b0r = b0.reshape(1, D)
    b1r = b1.reshape(1, D)
    b2r = b2.reshape(1, D)

    agg_k = _make_agg_kernel((NBMAX, NB0, NB1))
    h = _tc_first(in_feat, W0, dego)
    p = agg_k(h, sdk, wkk)
    h = _tc_mid(p, degi, dego, b0r, W1)
    p = agg_k(h, sdk, wkk)
    h = _tc_mid(p, degi, dego, b1r, W2)
    p = agg_k(h, sdk, wkk)
    return _tc_final(p, degi, b2r)
